# Initial kernel scaffold; baseline (speedup 1.0000x reference)
import jax
import jax.numpy as jnp
from jax.experimental import pallas as pl


def kernel(batch, ego_feature, query_pos, query_sine_embed, kv_pos, edge_feat, node_feat, params):
    raise NotImplementedError("write your pallas kernel here")


# speedup vs baseline: 10.7351x; 10.7351x over previous
import functools

import jax
import jax.numpy as jnp
import numpy as np
from jax import lax
from jax.experimental import pallas as pl
from jax.experimental.pallas import tpu as pltpu
from jax.experimental.pallas import tpu_sc as plsc

D = 128
M = 6
B = 10000
E = 160000
H = 8
HD = D // H
DD = 2 * D

NW = 32
BB = 400
EB = 640
GC = 40
C1 = 256
C2 = 64
NSEG_MAX = B // NW + 2


def _ln_rows(x, g, bt):
    mu = jnp.mean(x, axis=-1, keepdims=True)
    xc = x - mu
    var = jnp.mean(xc * xc, axis=-1, keepdims=True)
    return xc * jax.lax.rsqrt(var + 1e-5) * g + bt


def _sel_dh():
    di = jax.lax.broadcasted_iota(jnp.int32, (D, H), 0)
    hi = jax.lax.broadcasted_iota(jnp.int32, (D, H), 1)
    return (di // HD == hi).astype(jnp.float32)


def _sel_hd():
    hi = jax.lax.broadcasted_iota(jnp.int32, (H, D), 0)
    di = jax.lax.broadcasted_iota(jnp.int32, (H, D), 1)
    return (di // HD == hi).astype(jnp.float32)


def _qside_body(ego_ref, qpos_ref, qse_ref,
                sa_norm_g, sa_norm_bt, sa_qc_w, sa_qc_b, sa_qp_w, sa_qp_b,
                sa_kc_w, sa_kc_b, sa_kp_w, sa_kp_b, sa_v_w, sa_v_b,
                wq, bq, wk, bk, wv, bv, wo, bo,
                n1_w, n1_b, qp2_w, qp2_b, caq_w, caq_b, qsin_w, qsin_b,
                qcat_ref, egon_ref):
    (sa_norm_g, sa_norm_bt, sa_qc_w, sa_qc_b, sa_qp_w, sa_qp_b,
     sa_kc_w, sa_kc_b, sa_kp_w, sa_kp_b, sa_v_w, sa_v_b,
     wq, bq, wk, bk, wv, bv, wo, bo,
     n1_w, n1_b, qp2_w, qp2_b, caq_w, caq_b, qsin_w, qsin_b) = [
        r[...] for r in (
            sa_norm_g, sa_norm_bt, sa_qc_w, sa_qc_b, sa_qp_w, sa_qp_b,
            sa_kc_w, sa_kc_b, sa_kp_w, sa_kp_b, sa_v_w, sa_v_b,
            wq, bq, wk, bk, wv, bv, wo, bo,
            n1_w, n1_b, qp2_w, qp2_b, caq_w, caq_b, qsin_w, qsin_b)]
    sel = _sel_dh()
    selt = _sel_hd()
    qh, kh, vh, e_res = [], [], [], []
    for m in range(M):
        e_m = ego_ref[m]
        qp_m = qpos_ref[m]
        tgt = _ln_rows(e_m, sa_norm_g[0], sa_norm_bt[0])
        q_m = (jnp.dot(tgt, sa_qc_w.T, preferred_element_type=jnp.float32)
               + sa_qc_b[0]
               + jnp.dot(qp_m, sa_qp_w.T, preferred_element_type=jnp.float32)
               + sa_qp_b[0])
        k_m = (jnp.dot(tgt, sa_kc_w.T, preferred_element_type=jnp.float32)
               + sa_kc_b[0]
               + jnp.dot(qp_m, sa_kp_w.T, preferred_element_type=jnp.float32)
               + sa_kp_b[0])
        v_m = jnp.dot(tgt, sa_v_w.T, preferred_element_type=jnp.float32) + sa_v_b[0]
        qh.append(jnp.dot(q_m, wq.T, preferred_element_type=jnp.float32) + bq[0])
        kh.append(jnp.dot(k_m, wk.T, preferred_element_type=jnp.float32) + bk[0])
        vh.append(jnp.dot(v_m, wv.T, preferred_element_type=jnp.float32) + bv[0])
        e_res.append(e_m)

    inv_sqrt = 1.0 / np.sqrt(float(HD))
    for i in range(M):
        s_ij = [jnp.dot(qh[i] * kh[j], sel,
                        preferred_element_type=jnp.float32) * inv_sqrt
                for j in range(M)]
        mx = s_ij[0]
        for j in range(1, M):
            mx = jnp.maximum(mx, s_ij[j])
        ex = [jnp.exp(s - mx) for s in s_ij]
        sm = ex[0]
        for j in range(1, M):
            sm = sm + ex[j]
        inv = 1.0 / sm
        o_i = None
        for j in range(M):
            a_bc = jnp.dot(ex[j] * inv, selt, preferred_element_type=jnp.float32)
            term = a_bc * vh[j]
            o_i = term if o_i is None else o_i + term
        attn = jnp.dot(o_i, wo.T, preferred_element_type=jnp.float32) + bo[0]
        ego2 = e_res[i] + attn
        egon = jnp.dot(ego2, n1_w.T, preferred_element_type=jnp.float32) + n1_b[0]
        qp2 = (jnp.dot(qpos_ref[i], qp2_w.T, preferred_element_type=jnp.float32)
               + qp2_b[0])
        qq = (jnp.dot(egon + qp2, caq_w.T, preferred_element_type=jnp.float32)
              + caq_b[0])
        qse2 = (jnp.dot(qse_ref[:, i, :], qsin_w.T,
                        preferred_element_type=jnp.float32) + qsin_b[0])
        egon_ref[:, i, :] = egon
        qcat_ref[:, i, 0:D] = qq
        qcat_ref[:, i, D:DD] = qse2


def _qside_call(ego, qpos, qse, wlist):
    nb = B // BB
    w_specs = []
    for w in wlist:
        w_specs.append(pl.BlockSpec(w.shape, (lambda i, _r=len(w.shape): (0,) * _r)))
    in_specs = [
        pl.BlockSpec((M, BB, D), lambda i: (0, i, 0)),
        pl.BlockSpec((M, BB, D), lambda i: (0, i, 0)),
        pl.BlockSpec((BB, M, D), lambda i: (i, 0, 0)),
    ] + w_specs
    out_specs = [
        pl.BlockSpec((BB, M, DD), lambda i: (i, 0, 0)),
        pl.BlockSpec((BB, M, D), lambda i: (i, 0, 0)),
    ]
    return pl.pallas_call(
        _qside_body,
        grid=(nb,),
        in_specs=in_specs,
        out_specs=out_specs,
        out_shape=[
            jax.ShapeDtypeStruct((B, M, DD), jnp.float32),
            jax.ShapeDtypeStruct((B, M, D), jnp.float32),
        ],
        compiler_params=pltpu.CompilerParams(
            dimension_semantics=("arbitrary",)),
    )(ego, qpos, qse, *wlist)


def _gather_call(qtab, batch_i32):
    rows_per_w = E // NW
    nch = rows_per_w // GC
    mesh = plsc.VectorSubcoreMesh(core_axis_name="c", subcore_axis_name="s")

    @functools.partial(
        pl.kernel, mesh=mesh,
        out_type=jax.ShapeDtypeStruct((E, M * DD), jnp.float32),
        scratch_types=[
            pltpu.VMEM((GC,), jnp.int32),
            pltpu.VMEM((GC, M * DD), jnp.float32),
            pltpu.SemaphoreType.DMA,
        ],
    )
    def g_kernel(tab_hbm, idx_hbm, out_hbm, idx_v, rows_v, sem):
        wid = lax.axis_index("s") * 2 + lax.axis_index("c")
        base0 = wid * rows_per_w

        def chunk(j, carry):
            base = base0 + j * GC
            pltpu.sync_copy(idx_hbm.at[pl.ds(base, GC)], idx_v)
            pltpu.async_copy(tab_hbm.at[idx_v], rows_v, sem).wait()
            pltpu.sync_copy(rows_v, out_hbm.at[pl.ds(base, GC)])
            return carry

        lax.fori_loop(0, nch, chunk, 0)

    return g_kernel(qtab, batch_i32)


def _edge_body(ef_ref, nf_ref, kv_ref, qce_ref, b2_ref,
               nv1_g, nv1_bt, nv2_g, nv2_bt,
               cak_w, cak_b, kpos_w, kpos_b, v1_w, v1_b, v2_w, v2_b,
               vv_ref, sc_ref, st_ref):
    (nv1_g, nv1_bt, nv2_g, nv2_bt, cak_w, cak_b, kpos_w, kpos_b,
     v1_w, v1_b, v2_w, v2_b) = [
        r[...] for r in (nv1_g, nv1_bt, nv2_g, nv2_bt, cak_w, cak_b,
                         kpos_w, kpos_b, v1_w, v1_b, v2_w, v2_b)]
    i = pl.program_id(0)
    sel = _sel_dh()

    nfn = _ln_rows(nf_ref[...], nv2_g[0], nv2_bt[0])
    v2 = jnp.dot(nfn, v2_w.T, preferred_element_type=jnp.float32) + v2_b[0]
    kpos = jnp.dot(kv_ref[...], kpos_w.T, preferred_element_type=jnp.float32) + kpos_b[0]

    blk_max = None
    for m in range(M):
        efn = _ln_rows(ef_ref[:, m, :], nv1_g[0], nv1_bt[0])
        kk = jnp.dot(efn, cak_w.T, preferred_element_type=jnp.float32) + cak_b[0]
        v1 = jnp.dot(efn, v1_w.T, preferred_element_type=jnp.float32) + v1_b[0]
        vv_ref[:, m, :] = v1 + v2
        prod = kk * qce_ref[:, m, 0:D] + kpos * qce_ref[:, m, D:DD]
        s_m = jnp.dot(prod, sel, preferred_element_type=jnp.float32)
        sc_ref[:, m * H:(m + 1) * H] = s_m
        bm = jnp.max(s_m)
        blk_max = bm if blk_max is None else jnp.maximum(blk_max, bm)

    li = jax.lax.broadcasted_iota(jnp.int32, (1, D), 1)
    thr = (jnp.minimum(li, NW) * B) // NW
    bvals = b2_ref[...]
    cnt = jnp.sum((bvals < thr).astype(jnp.float32), axis=0, keepdims=True)

    @pl.when(i == 0)
    def _init():
        st_ref[...] = jnp.full((8, D), -1e30, jnp.float32)
        st_ref[0:1, :] = jnp.zeros((1, D), jnp.float32)

    st_ref[0:1, :] = st_ref[0:1, :] + cnt
    st_ref[1:2, :] = jnp.maximum(st_ref[1:2, :],
                                 jnp.full((1, D), 1.0, jnp.float32) * blk_max)


def _edge_call(ef, nf, kv, qce, batch2d, wlist):
    nb = E // EB
    w_specs = [pl.BlockSpec(w.shape, (lambda i, _r=len(w.shape): (0,) * _r))
               for w in wlist]
    in_specs = [
        pl.BlockSpec((EB, M, D), lambda i: (i, 0, 0)),
        pl.BlockSpec((EB, D), lambda i: (i, 0)),
        pl.BlockSpec((EB, D), lambda i: (i, 0)),
        pl.BlockSpec((EB, M, DD), lambda i: (i, 0, 0)),
        pl.BlockSpec((EB, 1), lambda i: (i, 0)),
    ] + w_specs
    out_specs = [
        pl.BlockSpec((EB, M, D), lambda i: (i, 0, 0)),
        pl.BlockSpec((EB, M * H), lambda i: (i, 0)),
        pl.BlockSpec((8, D), lambda i: (0, 0)),
    ]
    return pl.pallas_call(
        _edge_body,
        grid=(nb,),
        in_specs=in_specs,
        out_specs=out_specs,
        out_shape=[
            jax.ShapeDtypeStruct((E, M, D), jnp.float32),
            jax.ShapeDtypeStruct((E, M * H), jnp.float32),
            jax.ShapeDtypeStruct((8, D), jnp.float32),
        ],
        compiler_params=pltpu.CompilerParams(
            dimension_semantics=("arbitrary",)),
    )(ef, nf, kv, qce, batch2d, *wlist)


def _segment_call(batch_i32, scores, vv, starts_i32, gmax_f32):
    FD = M * D
    NCH = M * H
    mesh = plsc.VectorSubcoreMesh(core_axis_name="c", subcore_axis_name="s")

    @functools.partial(
        pl.kernel, mesh=mesh,
        out_type=jax.ShapeDtypeStruct((B * FD,), jnp.float32),
        scratch_types=[
            pltpu.VMEM((64,), jnp.int32),
            pltpu.VMEM((16,), jnp.float32),
            pltpu.VMEM((C1 * NCH,), jnp.float32),
            pltpu.VMEM((C1 + 16,), jnp.int32),
            pltpu.VMEM((NSEG_MAX * NCH,), jnp.float32),
            pltpu.VMEM((C2 * NCH,), jnp.float32),
            pltpu.VMEM((C2 + 16,), jnp.int32),
            pltpu.VMEM((C2 * FD,), jnp.float32),
            pltpu.VMEM((FD,), jnp.float32),
            pltpu.VMEM((NCH + 16,), jnp.float32),
            pltpu.VMEM((16 * FD,), jnp.float32),
        ],
    )
    def s_kernel(b_hbm, sc_hbm, vv_hbm, st_hbm, gm_hbm, agg_hbm,
                 st_v, gm_v, s1_v, b1_v, ssum_v, s2_v, b2_v, vv_v,
                 arow_v, alpha_v, z_v):
        wid = lax.axis_index("s") * 2 + lax.axis_index("c")
        b0 = (wid * B) // NW
        b1 = ((wid + 1) * B) // NW
        nseg = b1 - b0

        pltpu.sync_copy(st_hbm.at[pl.ds(0, 40)], st_v.at[pl.ds(0, 40)])
        pltpu.sync_copy(gm_hbm.at[pl.ds(0, 16)], gm_v)
        start = st_v[pl.ds(wid, 16)][0]
        end = st_v[pl.ds(wid + 1, 16)][0]
        gvec = gm_v[...]

        zv = jnp.zeros((16,), jnp.float32)

        def zfill(j, c):
            z_v[pl.ds(j * 16, 16)] = zv
            return c
        lax.fori_loop(0, FD, zfill, 0)

        nzb = nseg // 16

        def zout(j, c):
            pltpu.sync_copy(z_v, agg_hbm.at[pl.ds((b0 + j * 16) * FD, 16 * FD)])
            return c
        lax.fori_loop(0, nzb, zout, 0)
        rem = nseg - nzb * 16

        def zt(j, c):
            pltpu.sync_copy(
                z_v.at[pl.ds(0, FD)],
                agg_hbm.at[pl.ds((b0 + nzb * 16 + j) * FD, FD)])
            return c
        lax.fori_loop(0, rem, zt, 0)

        ci0 = start // C1
        ci1 = (end + C1 - 1) // C1

        def p1_chunk(ci, carry):
            cur, a0, a1, a2 = carry
            base = ci * C1
            pltpu.sync_copy(sc_hbm.at[pl.ds(base * NCH, C1 * NCH)], s1_v)
            pltpu.sync_copy(b_hbm.at[pl.ds(base, C1)], b1_v.at[pl.ds(0, C1)])
            lo = jnp.maximum(start - base, 0)
            hi = jnp.minimum(end - base, C1)

            def p1_edge(i, c):
                cur, a0, a1, a2 = c
                seg = b1_v[pl.ds(i, 16)][0]
                is_new = seg != cur

                @pl.when(jnp.logical_and(is_new, cur >= 0))
                def _flush():
                    r = (cur - b0) * NCH
                    ssum_v[pl.ds(r, 16)] = a0
                    ssum_v[pl.ds(r + 16, 16)] = a1
                    ssum_v[pl.ds(r + 32, 16)] = a2

                sb = i * NCH
                e0 = jnp.exp(s1_v[pl.ds(sb, 16)] - gvec)
                e1 = jnp.exp(s1_v[pl.ds(sb + 16, 16)] - gvec)
                e2 = jnp.exp(s1_v[pl.ds(sb + 32, 16)] - gvec)
                a0 = jnp.where(is_new, e0, a0 + e0)
                a1 = jnp.where(is_new, e1, a1 + e1)
                a2 = jnp.where(is_new, e2, a2 + e2)
                return (seg, a0, a1, a2)

            return lax.fori_loop(lo, hi, p1_edge, (cur, a0, a1, a2))

        cur, a0, a1, a2 = lax.fori_loop(
            ci0, ci1, p1_chunk, (jnp.int32(-1), zv, zv, zv))

        @pl.when(cur >= 0)
        def _flush_last():
            r = (cur - b0) * NCH
            ssum_v[pl.ds(r, 16)] = a0
            ssum_v[pl.ds(r + 16, 16)] = a1
            ssum_v[pl.ds(r + 32, 16)] = a2

        def zrow48(j, c):
            arow_v[pl.ds(j * 16, 16)] = zv
            return c
        lax.fori_loop(0, FD // 16, zrow48, 0)

        cj0 = start // C2
        cj1 = (end + C2 - 1) // C2

        def p2_chunk(cj, carry):
            cur, r0, r1, r2 = carry
            base = cj * C2
            pltpu.sync_copy(sc_hbm.at[pl.ds(base * NCH, C2 * NCH)], s2_v)
            pltpu.sync_copy(b_hbm.at[pl.ds(base, C2)], b2_v.at[pl.ds(0, C2)])
            pltpu.sync_copy(vv_hbm.at[pl.ds(base * FD, C2 * FD)], vv_v)
            lo = jnp.maximum(start - base, 0)
            hi = jnp.minimum(end - base, C2)

            def p2_edge(i, c):
                cur, r0, r1, r2 = c
                seg = b2_v[pl.ds(i, 16)][0]
                is_new = seg != cur

                @pl.when(jnp.logical_and(is_new, cur >= 0))
                def _flush():
                    pltpu.sync_copy(arow_v, agg_hbm.at[pl.ds(cur * FD, FD)])

                    def zr(j, cc):
                        arow_v[pl.ds(j * 16, 16)] = zv
                        return cc
                    lax.fori_loop(0, FD // 16, zr, 0)

                r = (seg - b0) * NCH
                n0 = jnp.where(is_new, 1.0 / ssum_v[pl.ds(r, 16)], r0)
                n1 = jnp.where(is_new, 1.0 / ssum_v[pl.ds(r + 16, 16)], r1)
                n2 = jnp.where(is_new, 1.0 / ssum_v[pl.ds(r + 32, 16)], r2)
                sb = i * NCH
                al0 = jnp.exp(s2_v[pl.ds(sb, 16)] - gvec) * n0
                al1 = jnp.exp(s2_v[pl.ds(sb + 16, 16)] - gvec) * n1
                al2 = jnp.exp(s2_v[pl.ds(sb + 32, 16)] - gvec) * n2
                alpha_v[pl.ds(0, 16)] = al0
                alpha_v[pl.ds(16, 16)] = al1
                alpha_v[pl.ds(32, 16)] = al2

                vb = i * FD

                def chan(j, cc):
                    a = alpha_v[pl.ds(j, 16)][0]
                    av = jnp.full((16,), a, jnp.float32)
                    sl = pl.ds(j * 16, 16)
                    arow_v[sl] = arow_v[sl] + av * vv_v[pl.ds(vb + j * 16, 16)]
                    return cc
                lax.fori_loop(0, NCH, chan, 0)
                return (seg, n0, n1, n2)

            return lax.fori_loop(lo, hi, p2_edge, (cur, r0, r1, r2))

        cur2, _, _, _ = lax.fori_loop(
            cj0, cj1, p2_chunk, (jnp.int32(-1), zv, zv, zv))

        @pl.when(cur2 >= 0)
        def _flush_last2():
            pltpu.sync_copy(arow_v, agg_hbm.at[pl.ds(cur2 * FD, FD)])

    return s_kernel(batch_i32, scores.reshape(-1), vv.reshape(-1),
                    starts_i32, gmax_f32)


def _final_body(egon_ref, agg_ref,
                n2_w, n2_b, l1_w, l1_b, l2_w, l2_b, out_ref):
    (n2_w, n2_b, l1_w, l1_b, l2_w, l2_b) = [
        r[...] for r in (n2_w, n2_b, l1_w, l1_b, l2_w, l2_b)]
    for m in range(M):
        o = egon_ref[:, m, :] + agg_ref[:, m, :]
        t = jnp.dot(o, n2_w.T, preferred_element_type=jnp.float32) + n2_b[0]
        t = jnp.dot(t, l1_w.T, preferred_element_type=jnp.float32) + l1_b[0]
        t = jnp.maximum(t, 0.0)
        t = jnp.dot(t, l2_w.T, preferred_element_type=jnp.float32) + l2_b[0]
        out_ref[m] = o + t


def _final_call(egon, agg, wlist):
    nb = B // BB
    w_specs = [pl.BlockSpec(w.shape, (lambda i, _r=len(w.shape): (0,) * _r))
               for w in wlist]
    in_specs = [
        pl.BlockSpec((BB, M, D), lambda i: (i, 0, 0)),
        pl.BlockSpec((BB, M, D), lambda i: (i, 0, 0)),
    ] + w_specs
    out_specs = pl.BlockSpec((M, BB, D), lambda i: (0, i, 0))
    return pl.pallas_call(
        _final_body,
        grid=(nb,),
        in_specs=in_specs,
        out_specs=out_specs,
        out_shape=jax.ShapeDtypeStruct((M, B, D), jnp.float32),
        compiler_params=pltpu.CompilerParams(
            dimension_semantics=("arbitrary",)),
    )(egon, agg, *wlist)


def kernel(batch, ego_feature, query_pos, query_sine_embed, kv_pos,
           edge_feat, node_feat, params):
    p = params
    r1 = lambda a: a.reshape(1, -1)

    wq = p['mha_in_w'][0:D]
    wk = p['mha_in_w'][D:2 * D]
    wv = p['mha_in_w'][2 * D:3 * D]
    bq = r1(p['mha_in_b'][0:D])
    bk = r1(p['mha_in_b'][D:2 * D])
    bv = r1(p['mha_in_b'][2 * D:3 * D])

    wlist_a = [
        r1(p['sa_norm_g']), r1(p['sa_norm_bt']),
        p['sa_qc_w'], r1(p['sa_qc_b']), p['sa_qp_w'], r1(p['sa_qp_b']),
        p['sa_kc_w'], r1(p['sa_kc_b']), p['sa_kp_w'], r1(p['sa_kp_b']),
        p['sa_v_w'], r1(p['sa_v_b']),
        wq, bq, wk, bk, wv, bv, p['mha_out_w'], r1(p['mha_out_b']),
        p['norm1_w'], r1(p['norm1_b']),
        p['ca_qpos_w'], r1(p['ca_qpos_b']),
        p['ca_q_w'], r1(p['ca_q_b']),
        p['ca_qsin_w'], r1(p['ca_qsin_b']),
    ]
    qcat, egon = _qside_call(ego_feature, query_pos, query_sine_embed, wlist_a)

    batch_i32 = batch.astype(jnp.int32)
    qtab = qcat.reshape(B, M * DD)
    qcat_edge = _gather_call(qtab, batch_i32)
    qce = qcat_edge.reshape(E, M, DD)

    wlist_b = [
        r1(p['norm_v1_g']), r1(p['norm_v1_bt']),
        r1(p['norm_v2_g']), r1(p['norm_v2_bt']),
        p['ca_k_w'], r1(p['ca_k_b']),
        p['ca_kpos_w'], r1(p['ca_kpos_b']),
        p['lin_v1_w'], r1(p['lin_v1_b']),
        p['lin_v2_w'], r1(p['lin_v2_b']),
    ]
    batch2d = batch_i32.reshape(E, 1)
    vv, scores, stats = _edge_call(edge_feat, node_feat, kv_pos, qce,
                                   batch2d, wlist_b)

    starts_i32 = stats[0].astype(jnp.int32)
    gmax = stats[1]
    vv2 = vv.reshape(E, M * D)
    agg2 = _segment_call(batch_i32, scores, vv2, starts_i32, gmax)
    agg = agg2.reshape(B, M, D)

    wlist_f = [
        p['norm2_w'], r1(p['norm2_b']),
        p['linear1_w'], r1(p['linear1_b']),
        p['linear2_w'], r1(p['linear2_b']),
    ]
    return _final_call(egon, agg, wlist_f)


# speedup vs baseline: 15.9133x; 1.4824x over previous
import functools

import jax
import jax.numpy as jnp
import numpy as np
from jax import lax
from jax.experimental import pallas as pl
from jax.experimental.pallas import tpu as pltpu
from jax.experimental.pallas import tpu_sc as plsc

D = 128
M = 6
B = 10000
E = 160000
H = 8
HD = D // H
DD = 2 * D

NW = 32
BB = 400
EB = 640
GC = 40
C1 = 256
C2 = 64
NSEG_MAX = B // NW + 2


def _ln_rows(x, g, bt):
    mu = jnp.mean(x, axis=-1, keepdims=True)
    xc = x - mu
    var = jnp.mean(xc * xc, axis=-1, keepdims=True)
    return xc * jax.lax.rsqrt(var + 1e-5) * g + bt


def _sel_dh():
    di = jax.lax.broadcasted_iota(jnp.int32, (D, H), 0)
    hi = jax.lax.broadcasted_iota(jnp.int32, (D, H), 1)
    return (di // HD == hi).astype(jnp.float32)


def _sel_hd():
    hi = jax.lax.broadcasted_iota(jnp.int32, (H, D), 0)
    di = jax.lax.broadcasted_iota(jnp.int32, (H, D), 1)
    return (di // HD == hi).astype(jnp.float32)


def _qside_body(ego_ref, qpos_ref, qse_ref,
                sa_norm_g, sa_norm_bt, sa_qc_w, sa_qc_b, sa_qp_w, sa_qp_b,
                sa_kc_w, sa_kc_b, sa_kp_w, sa_kp_b, sa_v_w, sa_v_b,
                wq, bq, wk, bk, wv, bv, wo, bo,
                n1_w, n1_b, qp2_w, qp2_b, caq_w, caq_b, qsin_w, qsin_b,
                qcat_ref, egon_ref):
    (sa_norm_g, sa_norm_bt, sa_qc_w, sa_qc_b, sa_qp_w, sa_qp_b,
     sa_kc_w, sa_kc_b, sa_kp_w, sa_kp_b, sa_v_w, sa_v_b,
     wq, bq, wk, bk, wv, bv, wo, bo,
     n1_w, n1_b, qp2_w, qp2_b, caq_w, caq_b, qsin_w, qsin_b) = [
        r[...] for r in (
            sa_norm_g, sa_norm_bt, sa_qc_w, sa_qc_b, sa_qp_w, sa_qp_b,
            sa_kc_w, sa_kc_b, sa_kp_w, sa_kp_b, sa_v_w, sa_v_b,
            wq, bq, wk, bk, wv, bv, wo, bo,
            n1_w, n1_b, qp2_w, qp2_b, caq_w, caq_b, qsin_w, qsin_b)]
    sel = _sel_dh()
    selt = _sel_hd()
    qh, kh, vh, e_res = [], [], [], []
    for m in range(M):
        e_m = ego_ref[m]
        qp_m = qpos_ref[m]
        tgt = _ln_rows(e_m, sa_norm_g[0], sa_norm_bt[0])
        q_m = (jnp.dot(tgt, sa_qc_w.T, preferred_element_type=jnp.float32)
               + sa_qc_b[0]
               + jnp.dot(qp_m, sa_qp_w.T, preferred_element_type=jnp.float32)
               + sa_qp_b[0])
        k_m = (jnp.dot(tgt, sa_kc_w.T, preferred_element_type=jnp.float32)
               + sa_kc_b[0]
               + jnp.dot(qp_m, sa_kp_w.T, preferred_element_type=jnp.float32)
               + sa_kp_b[0])
        v_m = jnp.dot(tgt, sa_v_w.T, preferred_element_type=jnp.float32) + sa_v_b[0]
        qh.append(jnp.dot(q_m, wq.T, preferred_element_type=jnp.float32) + bq[0])
        kh.append(jnp.dot(k_m, wk.T, preferred_element_type=jnp.float32) + bk[0])
        vh.append(jnp.dot(v_m, wv.T, preferred_element_type=jnp.float32) + bv[0])
        e_res.append(e_m)

    inv_sqrt = 1.0 / np.sqrt(float(HD))
    for i in range(M):
        s_ij = [jnp.dot(qh[i] * kh[j], sel,
                        preferred_element_type=jnp.float32) * inv_sqrt
                for j in range(M)]
        mx = s_ij[0]
        for j in range(1, M):
            mx = jnp.maximum(mx, s_ij[j])
        ex = [jnp.exp(s - mx) for s in s_ij]
        sm = ex[0]
        for j in range(1, M):
            sm = sm + ex[j]
        inv = 1.0 / sm
        o_i = None
        for j in range(M):
            a_bc = jnp.dot(ex[j] * inv, selt, preferred_element_type=jnp.float32)
            term = a_bc * vh[j]
            o_i = term if o_i is None else o_i + term
        attn = jnp.dot(o_i, wo.T, preferred_element_type=jnp.float32) + bo[0]
        ego2 = e_res[i] + attn
        egon = jnp.dot(ego2, n1_w.T, preferred_element_type=jnp.float32) + n1_b[0]
        qp2 = (jnp.dot(qpos_ref[i], qp2_w.T, preferred_element_type=jnp.float32)
               + qp2_b[0])
        qq = (jnp.dot(egon + qp2, caq_w.T, preferred_element_type=jnp.float32)
              + caq_b[0])
        qse2 = (jnp.dot(qse_ref[:, i, :], qsin_w.T,
                        preferred_element_type=jnp.float32) + qsin_b[0])
        egon_ref[:, i, :] = egon
        qcat_ref[:, i * DD:i * DD + D] = qq
        qcat_ref[:, i * DD + D:(i + 1) * DD] = qse2


def _qside_call(ego, qpos, qse, wlist):
    nb = B // BB
    w_specs = []
    for w in wlist:
        w_specs.append(pl.BlockSpec(w.shape, (lambda i, _r=len(w.shape): (0,) * _r)))
    in_specs = [
        pl.BlockSpec((M, BB, D), lambda i: (0, i, 0)),
        pl.BlockSpec((M, BB, D), lambda i: (0, i, 0)),
        pl.BlockSpec((BB, M, D), lambda i: (i, 0, 0)),
    ] + w_specs
    out_specs = [
        pl.BlockSpec((BB, M * DD), lambda i: (i, 0)),
        pl.BlockSpec((BB, M, D), lambda i: (i, 0, 0)),
    ]
    return pl.pallas_call(
        _qside_body,
        grid=(nb,),
        in_specs=in_specs,
        out_specs=out_specs,
        out_shape=[
            jax.ShapeDtypeStruct((B, M * DD), jnp.float32),
            jax.ShapeDtypeStruct((B, M, D), jnp.float32),
        ],
        compiler_params=pltpu.CompilerParams(
            dimension_semantics=("arbitrary",)),
    )(ego, qpos, qse, *wlist)


def _gather_call(qtab, batch_i32):
    rows_per_w = E // NW
    nch = rows_per_w // GC
    mesh = plsc.VectorSubcoreMesh(core_axis_name="c", subcore_axis_name="s")

    @functools.partial(
        pl.kernel, mesh=mesh,
        out_type=jax.ShapeDtypeStruct((E, M * DD), jnp.float32),
        scratch_types=[
            pltpu.VMEM((GC,), jnp.int32),
            pltpu.VMEM((GC, M * DD), jnp.float32),
            pltpu.SemaphoreType.DMA,
        ],
    )
    def g_kernel(tab_hbm, idx_hbm, out_hbm, idx_v, rows_v, sem):
        wid = lax.axis_index("s") * 2 + lax.axis_index("c")
        base0 = wid * rows_per_w

        def chunk(j, carry):
            base = base0 + j * GC
            pltpu.sync_copy(idx_hbm.at[pl.ds(base, GC)], idx_v)
            pltpu.async_copy(tab_hbm.at[idx_v], rows_v, sem).wait()
            pltpu.sync_copy(rows_v, out_hbm.at[pl.ds(base, GC)])
            return carry

        lax.fori_loop(0, nch, chunk, 0)

    return g_kernel(qtab, batch_i32)


def _edge_body(ef_ref, nf_ref, kv_ref, qce_ref, b2_ref,
               nv1_g, nv1_bt, nv2_g, nv2_bt,
               cak_w, cak_b, kpos_w, kpos_b, v1_w, v1_b, v2_w, v2_b,
               vv_ref, sc_ref, st_ref):
    (nv1_g, nv1_bt, nv2_g, nv2_bt, cak_w, cak_b, kpos_w, kpos_b,
     v1_w, v1_b, v2_w, v2_b) = [
        r[...] for r in (nv1_g, nv1_bt, nv2_g, nv2_bt, cak_w, cak_b,
                         kpos_w, kpos_b, v1_w, v1_b, v2_w, v2_b)]
    i = pl.program_id(0)
    sel = _sel_dh()

    nfn = _ln_rows(nf_ref[...], nv2_g[0], nv2_bt[0])
    v2 = jnp.dot(nfn, v2_w.T, preferred_element_type=jnp.float32) + v2_b[0]
    kpos = jnp.dot(kv_ref[...], kpos_w.T, preferred_element_type=jnp.float32) + kpos_b[0]

    blk_max = None
    for m in range(M):
        efn = _ln_rows(ef_ref[:, m, :], nv1_g[0], nv1_bt[0])
        kk = jnp.dot(efn, cak_w.T, preferred_element_type=jnp.float32) + cak_b[0]
        v1 = jnp.dot(efn, v1_w.T, preferred_element_type=jnp.float32) + v1_b[0]
        vv_ref[:, m * D:(m + 1) * D] = v1 + v2
        prod = (kk * qce_ref[:, m * DD:m * DD + D]
                + kpos * qce_ref[:, m * DD + D:(m + 1) * DD])
        s_m = jnp.dot(prod, sel, preferred_element_type=jnp.float32)
        sc_ref[:, m * H:(m + 1) * H] = s_m
        bm = jnp.max(s_m)
        blk_max = bm if blk_max is None else jnp.maximum(blk_max, bm)

    li = jax.lax.broadcasted_iota(jnp.int32, (1, D), 1)
    thr = (jnp.minimum(li, NW) * B) // NW
    bvals = b2_ref[...]
    cnt = jnp.sum((bvals < thr).astype(jnp.float32), axis=0, keepdims=True)

    @pl.when(i == 0)
    def _init():
        st_ref[...] = jnp.full((8, D), -1e30, jnp.float32)
        st_ref[0:1, :] = jnp.zeros((1, D), jnp.float32)

    st_ref[0:1, :] = st_ref[0:1, :] + cnt
    st_ref[1:2, :] = jnp.maximum(st_ref[1:2, :],
                                 jnp.full((1, D), 1.0, jnp.float32) * blk_max)


def _edge_call(ef, nf, kv, qce, batch2d, wlist):
    nb = E // EB
    w_specs = [pl.BlockSpec(w.shape, (lambda i, _r=len(w.shape): (0,) * _r))
               for w in wlist]
    in_specs = [
        pl.BlockSpec((EB, M, D), lambda i: (i, 0, 0)),
        pl.BlockSpec((EB, D), lambda i: (i, 0)),
        pl.BlockSpec((EB, D), lambda i: (i, 0)),
        pl.BlockSpec((EB, M * DD), lambda i: (i, 0)),
        pl.BlockSpec((EB, 1), lambda i: (i, 0)),
    ] + w_specs
    out_specs = [
        pl.BlockSpec((EB, M * D), lambda i: (i, 0)),
        pl.BlockSpec((EB, M * H), lambda i: (i, 0)),
        pl.BlockSpec((8, D), lambda i: (0, 0)),
    ]
    return pl.pallas_call(
        _edge_body,
        grid=(nb,),
        in_specs=in_specs,
        out_specs=out_specs,
        out_shape=[
            jax.ShapeDtypeStruct((E, M * D), jnp.float32),
            jax.ShapeDtypeStruct((E, M * H), jnp.float32),
            jax.ShapeDtypeStruct((8, D), jnp.float32),
        ],
        compiler_params=pltpu.CompilerParams(
            dimension_semantics=("arbitrary",)),
    )(ef, nf, kv, qce, batch2d, *wlist)


def _segment_call(batch_i32, scores, vv, starts_i32, gmax_f32):
    FD = M * D
    NCH = M * H
    mesh = plsc.VectorSubcoreMesh(core_axis_name="c", subcore_axis_name="s")

    @functools.partial(
        pl.kernel, mesh=mesh,
        out_type=jax.ShapeDtypeStruct((B * FD,), jnp.float32),
        scratch_types=[
            pltpu.VMEM((64,), jnp.int32),
            pltpu.VMEM((16,), jnp.float32),
            pltpu.VMEM((C1, NCH), jnp.float32),
            pltpu.VMEM((C1 + 16,), jnp.int32),
            pltpu.VMEM((NSEG_MAX * NCH,), jnp.float32),
            pltpu.VMEM((C2, NCH), jnp.float32),
            pltpu.VMEM((C2 + 16,), jnp.int32),
            pltpu.VMEM((C2, FD), jnp.float32),
            pltpu.VMEM((FD,), jnp.float32),
            pltpu.VMEM((NCH + 16,), jnp.float32),
            pltpu.VMEM((16 * FD,), jnp.float32),
        ],
    )
    def s_kernel(b_hbm, sc_hbm, vv_hbm, st_hbm, gm_hbm, agg_hbm,
                 st_v, gm_v, s1_v, b1_v, ssum_v, s2_v, b2_v, vv_v,
                 arow_v, alpha_v, z_v):
        wid = lax.axis_index("s") * 2 + lax.axis_index("c")
        b0 = (wid * B) // NW
        b1 = ((wid + 1) * B) // NW
        nseg = b1 - b0

        pltpu.sync_copy(st_hbm.at[pl.ds(0, 40)], st_v.at[pl.ds(0, 40)])
        pltpu.sync_copy(gm_hbm.at[pl.ds(0, 16)], gm_v)
        start = st_v[pl.ds(wid, 16)][0]
        end = st_v[pl.ds(wid + 1, 16)][0]
        gvec = gm_v[...]

        zv = jnp.zeros((16,), jnp.float32)

        def zfill(j, c):
            z_v[pl.ds(j * 16, 16)] = zv
            return c
        lax.fori_loop(0, FD, zfill, 0)

        nzb = nseg // 16

        def zout(j, c):
            pltpu.sync_copy(z_v, agg_hbm.at[pl.ds((b0 + j * 16) * FD, 16 * FD)])
            return c
        lax.fori_loop(0, nzb, zout, 0)
        rem = nseg - nzb * 16

        def zt(j, c):
            pltpu.sync_copy(
                z_v.at[pl.ds(0, FD)],
                agg_hbm.at[pl.ds((b0 + nzb * 16 + j) * FD, FD)])
            return c
        lax.fori_loop(0, rem, zt, 0)

        ci0 = start // C1
        ci1 = (end + C1 - 1) // C1

        def p1_chunk(ci, carry):
            cur, a0, a1, a2 = carry
            base = ci * C1
            pltpu.sync_copy(sc_hbm.at[pl.ds(base, C1)], s1_v)
            pltpu.sync_copy(b_hbm.at[pl.ds(base, C1)], b1_v.at[pl.ds(0, C1)])
            lo = jnp.maximum(start - base, 0)
            hi = jnp.minimum(end - base, C1)

            def p1_edge(i, c):
                cur, a0, a1, a2 = c
                seg = b1_v[pl.ds(i, 16)][0]
                is_new = seg != cur

                @pl.when(jnp.logical_and(is_new, cur >= 0))
                def _flush():
                    r = (cur - b0) * NCH
                    ssum_v[pl.ds(r, 16)] = a0
                    ssum_v[pl.ds(r + 16, 16)] = a1
                    ssum_v[pl.ds(r + 32, 16)] = a2

                e0 = jnp.exp(s1_v[i, pl.ds(0, 16)] - gvec)
                e1 = jnp.exp(s1_v[i, pl.ds(16, 16)] - gvec)
                e2 = jnp.exp(s1_v[i, pl.ds(32, 16)] - gvec)
                a0 = jnp.where(is_new, e0, a0 + e0)
                a1 = jnp.where(is_new, e1, a1 + e1)
                a2 = jnp.where(is_new, e2, a2 + e2)
                return (seg, a0, a1, a2)

            return lax.fori_loop(lo, hi, p1_edge, (cur, a0, a1, a2))

        cur, a0, a1, a2 = lax.fori_loop(
            ci0, ci1, p1_chunk, (jnp.int32(-1), zv, zv, zv))

        @pl.when(cur >= 0)
        def _flush_last():
            r = (cur - b0) * NCH
            ssum_v[pl.ds(r, 16)] = a0
            ssum_v[pl.ds(r + 16, 16)] = a1
            ssum_v[pl.ds(r + 32, 16)] = a2

        def zrow48(j, c):
            arow_v[pl.ds(j * 16, 16)] = zv
            return c
        lax.fori_loop(0, FD // 16, zrow48, 0)

        cj0 = start // C2
        cj1 = (end + C2 - 1) // C2

        def p2_chunk(cj, carry):
            cur, r0, r1, r2 = carry
            base = cj * C2
            pltpu.sync_copy(sc_hbm.at[pl.ds(base, C2)], s2_v)
            pltpu.sync_copy(b_hbm.at[pl.ds(base, C2)], b2_v.at[pl.ds(0, C2)])
            pltpu.sync_copy(vv_hbm.at[pl.ds(base, C2)], vv_v)
            lo = jnp.maximum(start - base, 0)
            hi = jnp.minimum(end - base, C2)

            def p2_edge(i, c):
                cur, r0, r1, r2 = c
                seg = b2_v[pl.ds(i, 16)][0]
                is_new = seg != cur

                @pl.when(jnp.logical_and(is_new, cur >= 0))
                def _flush():
                    pltpu.sync_copy(arow_v, agg_hbm.at[pl.ds(cur * FD, FD)])

                    def zr(j, cc):
                        arow_v[pl.ds(j * 16, 16)] = zv
                        return cc
                    lax.fori_loop(0, FD // 16, zr, 0)

                r = (seg - b0) * NCH
                n0 = jnp.where(is_new, 1.0 / ssum_v[pl.ds(r, 16)], r0)
                n1 = jnp.where(is_new, 1.0 / ssum_v[pl.ds(r + 16, 16)], r1)
                n2 = jnp.where(is_new, 1.0 / ssum_v[pl.ds(r + 32, 16)], r2)
                al0 = jnp.exp(s2_v[i, pl.ds(0, 16)] - gvec) * n0
                al1 = jnp.exp(s2_v[i, pl.ds(16, 16)] - gvec) * n1
                al2 = jnp.exp(s2_v[i, pl.ds(32, 16)] - gvec) * n2
                alpha_v[pl.ds(0, 16)] = al0
                alpha_v[pl.ds(16, 16)] = al1
                alpha_v[pl.ds(32, 16)] = al2

                def chan(j, cc):
                    a = alpha_v[pl.ds(j, 16)][0]
                    av = jnp.full((16,), a, jnp.float32)
                    sl = pl.ds(j * 16, 16)
                    arow_v[sl] = arow_v[sl] + av * vv_v[i, sl]
                    return cc
                lax.fori_loop(0, NCH, chan, 0)
                return (seg, n0, n1, n2)

            return lax.fori_loop(lo, hi, p2_edge, (cur, r0, r1, r2))

        cur2, _, _, _ = lax.fori_loop(
            cj0, cj1, p2_chunk, (jnp.int32(-1), zv, zv, zv))

        @pl.when(cur2 >= 0)
        def _flush_last2():
            pltpu.sync_copy(arow_v, agg_hbm.at[pl.ds(cur2 * FD, FD)])

    return s_kernel(batch_i32, scores, vv, starts_i32, gmax_f32)


def _final_body(egon_ref, agg_ref,
                n2_w, n2_b, l1_w, l1_b, l2_w, l2_b, out_ref):
    (n2_w, n2_b, l1_w, l1_b, l2_w, l2_b) = [
        r[...] for r in (n2_w, n2_b, l1_w, l1_b, l2_w, l2_b)]
    for m in range(M):
        o = egon_ref[:, m, :] + agg_ref[:, m, :]
        t = jnp.dot(o, n2_w.T, preferred_element_type=jnp.float32) + n2_b[0]
        t = jnp.dot(t, l1_w.T, preferred_element_type=jnp.float32) + l1_b[0]
        t = jnp.maximum(t, 0.0)
        t = jnp.dot(t, l2_w.T, preferred_element_type=jnp.float32) + l2_b[0]
        out_ref[m] = o + t


def _final_call(egon, agg, wlist):
    nb = B // BB
    w_specs = [pl.BlockSpec(w.shape, (lambda i, _r=len(w.shape): (0,) * _r))
               for w in wlist]
    in_specs = [
        pl.BlockSpec((BB, M, D), lambda i: (i, 0, 0)),
        pl.BlockSpec((BB, M, D), lambda i: (i, 0, 0)),
    ] + w_specs
    out_specs = pl.BlockSpec((M, BB, D), lambda i: (0, i, 0))
    return pl.pallas_call(
        _final_body,
        grid=(nb,),
        in_specs=in_specs,
        out_specs=out_specs,
        out_shape=jax.ShapeDtypeStruct((M, B, D), jnp.float32),
        compiler_params=pltpu.CompilerParams(
            dimension_semantics=("arbitrary",)),
    )(egon, agg, *wlist)


def kernel(batch, ego_feature, query_pos, query_sine_embed, kv_pos,
           edge_feat, node_feat, params):
    p = params
    r1 = lambda a: a.reshape(1, -1)

    wq = p['mha_in_w'][0:D]
    wk = p['mha_in_w'][D:2 * D]
    wv = p['mha_in_w'][2 * D:3 * D]
    bq = r1(p['mha_in_b'][0:D])
    bk = r1(p['mha_in_b'][D:2 * D])
    bv = r1(p['mha_in_b'][2 * D:3 * D])

    wlist_a = [
        r1(p['sa_norm_g']), r1(p['sa_norm_bt']),
        p['sa_qc_w'], r1(p['sa_qc_b']), p['sa_qp_w'], r1(p['sa_qp_b']),
        p['sa_kc_w'], r1(p['sa_kc_b']), p['sa_kp_w'], r1(p['sa_kp_b']),
        p['sa_v_w'], r1(p['sa_v_b']),
        wq, bq, wk, bk, wv, bv, p['mha_out_w'], r1(p['mha_out_b']),
        p['norm1_w'], r1(p['norm1_b']),
        p['ca_qpos_w'], r1(p['ca_qpos_b']),
        p['ca_q_w'], r1(p['ca_q_b']),
        p['ca_qsin_w'], r1(p['ca_qsin_b']),
    ]
    qcat, egon = _qside_call(ego_feature, query_pos, query_sine_embed, wlist_a)

    batch_i32 = batch.astype(jnp.int32)
    qcat_edge = _gather_call(qcat, batch_i32)

    wlist_b = [
        r1(p['norm_v1_g']), r1(p['norm_v1_bt']),
        r1(p['norm_v2_g']), r1(p['norm_v2_bt']),
        p['ca_k_w'], r1(p['ca_k_b']),
        p['ca_kpos_w'], r1(p['ca_kpos_b']),
        p['lin_v1_w'], r1(p['lin_v1_b']),
        p['lin_v2_w'], r1(p['lin_v2_b']),
    ]
    batch2d = batch_i32.reshape(E, 1)
    vv, scores, stats = _edge_call(edge_feat, node_feat, kv_pos, qcat_edge,
                                    batch2d, wlist_b)

    starts_i32 = stats[0].astype(jnp.int32)
    gmax = stats[1]
    agg2 = _segment_call(batch_i32, scores, vv, starts_i32, gmax)
    agg = agg2.reshape(B, M, D)

    wlist_f = [
        p['norm2_w'], r1(p['norm2_b']),
        p['linear1_w'], r1(p['linear1_b']),
        p['linear2_w'], r1(p['linear2_b']),
    ]
    return _final_call(egon, agg, wlist_f)
